# Initial kernel scaffold; baseline (speedup 1.0000x reference)
import jax
import jax.numpy as jnp
from jax.experimental import pallas as pl


def kernel(node_states, relations_0, relations_1, relations_2, w0_1, b0_1, w0_2, b0_2, w1_1, b1_1, w1_2, b1_2, w2_1, b2_1, w2_2, b2_2, wu1, bu1, wu2, bu2):
    raise NotImplementedError("write your pallas kernel here")


# speedup vs baseline: 1.0052x; 1.0052x over previous
import functools

import jax
import jax.numpy as jnp
from jax import lax
from jax.experimental import pallas as pl
from jax.experimental.pallas import tpu as pltpu
from jax.experimental.pallas import tpu_sc as plsc

H = 128
N_NODES = 50000
_ARITY = (1, 2, 3)
_E = (200000, 200000, 150000)
_EPAD_G = (204800, 204800, 159744)
_EPAD_S = (204800, 204800, 155648)
_SINK = N_NODES
_ACC_ROWS = N_NODES + 48

_MESH = dict(core_axis_name="c", subcore_axis_name="s", num_cores=2,
             num_subcores=16)


def _gather_sc(node_states, idx0, idx1, idx2):
    mesh = plsc.VectorSubcoreMesh(**_MESH)

    @functools.partial(
        pl.kernel,
        out_type=[jax.ShapeDtypeStruct((_EPAD_G[r], H), jnp.float32)
                  for r in range(3)],
        mesh=mesh,
        scratch_types=[
            pltpu.VMEM((128,), jnp.int32),
            pltpu.VMEM((128, H), jnp.float32),
            pltpu.SemaphoreType.DMA,
        ],
    )
    def k(ns_hbm, i0, i1, i2, g0, g1, g2, idx_v, rows_v, sem):
        w = lax.axis_index("s") * 2 + lax.axis_index("c")
        for r, (ih, gh) in enumerate(((i0, g0), (i1, g1), (i2, g2))):
            units = _EPAD_G[r] // (128 * 32)
            row0 = w * units

            def body(i, _, ih=ih, gh=gh, row0=row0):
                row = row0 + i
                pltpu.sync_copy(ih.at[pl.ds(row * 128, 128)], idx_v)
                pltpu.async_copy(ns_hbm.at[idx_v], rows_v, sem).wait()
                pltpu.sync_copy(rows_v, gh.at[pl.ds(row * 128, 128)])
                return 0

            lax.fori_loop(0, units, body, 0)

    return k(node_states, idx0, idx1, idx2)


def _mlp_tc(x, w1, b1, w2, b2, n_rows, block_rows):
    d = x.shape[1]
    grid = (n_rows // block_rows,)

    def body(x_ref, w1_ref, b1_ref, w2_ref, b2_ref, y_ref, mx_ref):
        h = jnp.maximum(
            jnp.dot(x_ref[...], w1_ref[...],
                    preferred_element_type=jnp.float32) + b1_ref[...], 0.0)
        y = jnp.dot(h, w2_ref[...],
                    preferred_element_type=jnp.float32) + b2_ref[...]
        y_ref[...] = y

        @pl.when(pl.program_id(0) == 0)
        def _init():
            mx_ref[0, 0] = -jnp.inf

        mx_ref[0, 0] = jnp.maximum(mx_ref[0, 0], jnp.max(y))

    return pl.pallas_call(
        body,
        grid=grid,
        in_specs=[
            pl.BlockSpec((block_rows, d), lambda i: (i, 0)),
            pl.BlockSpec((d, d), lambda i: (0, 0)),
            pl.BlockSpec((1, d), lambda i: (0, 0)),
            pl.BlockSpec((d, d), lambda i: (0, 0)),
            pl.BlockSpec((1, d), lambda i: (0, 0)),
        ],
        out_specs=[
            pl.BlockSpec((block_rows, d), lambda i: (i, 0)),
            pl.BlockSpec(memory_space=pltpu.SMEM),
        ],
        out_shape=[
            jax.ShapeDtypeStruct((n_rows, d), jnp.float32),
            jax.ShapeDtypeStruct((1, 1), jnp.float32),
        ],
    )(x, w1, b1.reshape(1, d), w2, b2.reshape(1, d))


def _exp_tc(y, m):
    e_rows = y.shape[0]
    bt = 512
    grid = (pl.cdiv(e_rows, bt),)

    def body(y_ref, m_ref, u_ref):
        u_ref[...] = jnp.exp(8.0 * (y_ref[...] - m_ref[0, 0]))

    return pl.pallas_call(
        body,
        grid=grid,
        in_specs=[
            pl.BlockSpec((bt, H), lambda i: (i, 0)),
            pl.BlockSpec(memory_space=pltpu.SMEM),
        ],
        out_specs=pl.BlockSpec((bt, H), lambda i: (i, 0)),
        out_shape=jax.ShapeDtypeStruct((e_rows, H), jnp.float32),
    )(y, m)


def _scatter_sc(u0, u1, u2, s0, s1, s2, zeros):
    mesh = plsc.VectorSubcoreMesh(**_MESH)

    @functools.partial(
        pl.kernel,
        out_type=jax.ShapeDtypeStruct((4, N_NODES, 32), jnp.float32),
        mesh=mesh,
        scratch_types=[
            pltpu.VMEM((4, 128), jnp.int32),
            pltpu.VMEM((4, 128), jnp.int32),
            pltpu.VMEM((4, 128, 32), jnp.float32),
            pltpu.VMEM_SHARED((_ACC_ROWS, 32), jnp.float32),
            pltpu.SemaphoreType.DMA,
        ],
        compiler_params=pltpu.CompilerParams(use_tc_tiling_on_sc=False),
    )
    def k(v0, v1, v2, n0, n1, n2, zr, out, nidx_v, ridx_v, vals_v, acc_sh,
          sem):
        c = lax.axis_index("c")
        s = lax.axis_index("s")
        iota = lax.iota(jnp.int32, 16)
        for qi in range(2):
            q = 2 * c + qi
            pltpu.sync_copy(zr.at[pl.ds(s * 3128, 3128)],
                            acc_sh.at[pl.ds(s * 3128, 3128)])
            plsc.subcore_barrier()
            for r, (vh, nh) in enumerate(((v0, n0), (v1, n1), (v2, n2))):
                per_tile = _EPAD_S[r] // 16
                n_chunks = per_tile // 512
                e_base = s * per_tile

                def body(i, _, vh=vh, nh=nh, e_base=e_base, e_max=_E[r] - 1):
                    e0 = e_base + i * 512
                    for j in range(4):
                        pltpu.sync_copy(nh.at[pl.ds(e0 + j * 128, 128)],
                                        nidx_v.at[j])
                        for t in range(8):
                            e_vec = jnp.minimum(e0 + j * 128 + t * 16 + iota,
                                                e_max)
                            ridx_v[j, pl.ds(t * 16, 16)] = e_vec * 4 + q
                    descs = [
                        pltpu.async_copy(vh.at[ridx_v.at[j]], vals_v.at[j],
                                         sem)
                        for j in range(4)
                    ]
                    for dsc in descs:
                        dsc.wait()
                    for j in range(4):
                        pltpu.sync_copy(vals_v.at[j],
                                        acc_sh.at[nidx_v.at[j]], add=True)
                    return 0

                lax.fori_loop(0, n_chunks, body, 0)
            plsc.subcore_barrier()
            @pl.when(s < 15)
            def _wb_full():
                pltpu.sync_copy(acc_sh.at[pl.ds(s * 3128, 3128)],
                                out.at[q, pl.ds(s * 3128, 3128)])

            @pl.when(s == 15)
            def _wb_last():
                pltpu.sync_copy(acc_sh.at[pl.ds(15 * 3128, 3080)],
                                out.at[q, pl.ds(15 * 3128, 3080)])

            plsc.subcore_barrier()

    return k(u0, u1, u2, s0, s1, s2, zeros)


def _update_tc(acc4, node_states, wu1a, wu1b, bu1, wu2, bu2, m):
    bt = 1000
    grid = (N_NODES // bt,)

    def body(a0, a1, a2, a3, ns_ref, w1a_ref, w1b_ref, b1_ref, w2_ref,
             b2_ref, m_ref, o_ref):
        h = jnp.dot(ns_ref[...], w1b_ref[...],
                    preferred_element_type=jnp.float32) + b1_ref[...]
        for q, a_ref in enumerate((a0, a1, a2, a3)):
            t = jnp.log(a_ref[0] + 1e-16) * 0.125 + m_ref[0, 0]
            h += jnp.dot(t, w1a_ref[pl.ds(q * 32, 32), :],
                         preferred_element_type=jnp.float32)
        h = jnp.maximum(h, 0.0)
        o_ref[...] = jnp.dot(h, w2_ref[...],
                             preferred_element_type=jnp.float32) + b2_ref[...]

    qspecs = [
        pl.BlockSpec((1, bt, 32), lambda i, q=q: (q, i, 0)) for q in range(4)
    ]
    return pl.pallas_call(
        body,
        grid=grid,
        in_specs=qspecs + [
            pl.BlockSpec((bt, H), lambda i: (i, 0)),
            pl.BlockSpec((H, 2 * H), lambda i: (0, 0)),
            pl.BlockSpec((H, 2 * H), lambda i: (0, 0)),
            pl.BlockSpec((1, 2 * H), lambda i: (0, 0)),
            pl.BlockSpec((2 * H, H), lambda i: (0, 0)),
            pl.BlockSpec((1, H), lambda i: (0, 0)),
            pl.BlockSpec(memory_space=pltpu.SMEM),
        ],
        out_specs=pl.BlockSpec((bt, H), lambda i: (i, 0)),
        out_shape=jax.ShapeDtypeStruct((N_NODES, H), jnp.float32),
    )(acc4, acc4, acc4, acc4, node_states, wu1a, wu1b,
      bu1.reshape(1, 2 * H), wu2, bu2.reshape(1, H), m)


def _pad_idx(rel, epad, fill):
    pad = epad - rel.shape[0]
    return jnp.concatenate([rel, jnp.full((pad,), fill, dtype=jnp.int32)])


def kernel(node_states, relations_0, relations_1, relations_2, w0_1, b0_1,
           w0_2, b0_2, w1_1, b1_1, w1_2, b1_2, w2_1, b2_1, w2_2, b2_2, wu1,
           bu1, wu2, bu2):
    rels = (relations_0, relations_1, relations_2)
    params = ((w0_1, b0_1, w0_2, b0_2), (w1_1, b1_1, w1_2, b1_2),
              (w2_1, b2_1, w2_2, b2_2))

    idx_g = [_pad_idx(rels[r], _EPAD_G[r], 0) for r in range(3)]
    g = _gather_sc(node_states, *idx_g)

    blocks = (2000, 1000, 1000)
    ys, ms = [], []
    for r in range(3):
        d = _ARITY[r] * H
        x = g[r].reshape(_EPAD_G[r] // _ARITY[r], d)
        y, m = _mlp_tc(x, params[r][0], params[r][1], params[r][2],
                       params[r][3], _E[r] // _ARITY[r], blocks[r])
        ys.append(y)
        ms.append(m)

    m_all = jnp.maximum(jnp.maximum(ms[0][0, 0], ms[1][0, 0]),
                        ms[2][0, 0]).reshape(1, 1)

    us = [_exp_tc(ys[r].reshape(_E[r], H), m_all) for r in range(3)]

    idx_s = [_pad_idx(rels[r], _EPAD_S[r], _SINK) for r in range(3)]
    zeros = jnp.zeros((_ACC_ROWS, 32), dtype=jnp.float32)
    acc = _scatter_sc(us[0].reshape(4 * _E[0], 32),
                      us[1].reshape(4 * _E[1], 32),
                      us[2].reshape(4 * _E[2], 32),
                      idx_s[0], idx_s[1], idx_s[2], zeros)

    return _update_tc(acc, node_states, wu1[:H], wu1[H:], bu1, wu2, bu2,
                      m_all)
